# baseline (speedup 1.0000x reference)
import functools

import jax
import jax.numpy as jnp
from jax import lax
from jax.experimental import pallas as pl
from jax.experimental.pallas import tpu as pltpu
from jax.experimental.pallas import tpu_sc as plsc

N_NODES = 10000
N_EDGES = 320000
D_FEAT = 128

NUM_CORES = 2
NUM_SUBCORES = 16
NUM_WORKERS = NUM_CORES * NUM_SUBCORES

CHUNK = 128
NCHUNKS = N_EDGES // CHUNK
BASE_CH = NCHUNKS // NUM_WORKERS
HALF = 40
N_PAD = 10240
ROWS_PER_TILE = N_PAD // NUM_SUBCORES

_mesh = plsc.VectorSubcoreMesh(core_axis_name="c", subcore_axis_name="s")


@functools.partial(
    pl.kernel,
    mesh=_mesh,
    out_type=jax.ShapeDtypeStruct((NUM_CORES, N_PAD, D_FEAT), jnp.float32),
    scratch_types=[
        pltpu.VMEM((2 * HALF,), jnp.int32),
        pltpu.VMEM((HALF, CHUNK), jnp.int32),
        pltpu.VMEM((HALF, CHUNK), jnp.int32),
        pltpu.VMEM((2, CHUNK, D_FEAT), jnp.float32),
        pltpu.VMEM_SHARED((N_PAD, D_FEAT), jnp.float32),
        pltpu.SemaphoreType.DMA,
        pltpu.SemaphoreType.DMA,
        pltpu.SemaphoreType.DMA,
        pltpu.SemaphoreType.DMA,
        pltpu.SemaphoreType.DMA,
    ],
)
def _mp_scatter(src_hbm, dst_hbm, x_hbm, zeros_hbm, out_hbm,
                ilist_v, src_v, dst_v, rows_v, acc_sh,
                sem0a, sem0b, sem1a, sem1b, semz):
    cid = lax.axis_index("c")
    sid = lax.axis_index("s")
    wid = sid * NUM_CORES + cid
    row0 = sid * ROWS_PER_TILE

    pltpu.async_copy(zeros_hbm, acc_sh.at[pl.ds(row0, ROWS_PER_TILE)], semz)

    lane = lax.iota(jnp.int32, 16)
    for g in range(2 * HALF // 16):
        ilist_v[pl.ds(16 * g, 16)] = jnp.minimum(
            wid + 32 * (16 * g) + 32 * lane, NCHUNKS - 1)

    sems = ((sem0a, sem0b), (sem1a, sem1b))
    HC = CHUNK // 2

    def fire(j, b):
        for k in range(2):
            pltpu.async_copy(x_hbm.at[src_v.at[j, pl.ds(k * HC, HC)]],
                             rows_v.at[b, pl.ds(k * HC, HC)], sems[b][k])

    def drain_scatter(j, b):
        for k in range(2):
            pltpu.make_async_copy(x_hbm.at[src_v.at[j, pl.ds(k * HC, HC)]],
                                  rows_v.at[b, pl.ds(k * HC, HC)],
                                  sems[b][k]).wait()
        pltpu.sync_copy(rows_v.at[b], acc_sh.at[dst_v.at[j]], add=True)

    for h in range(2):
        pltpu.sync_copy(src_hbm.at[ilist_v.at[pl.ds(h * HALF, HALF)]], src_v)
        pltpu.sync_copy(dst_hbm.at[ilist_v.at[pl.ds(h * HALF, HALF)]], dst_v)

        fire(0, 0)

        if h == 0:
            pltpu.make_async_copy(
                zeros_hbm, acc_sh.at[pl.ds(row0, ROWS_PER_TILE)], semz).wait()
            plsc.subcore_barrier()
            npairs = HALF // 2
        else:
            npairs = (BASE_CH - HALF) // 2

        def body(g, carry):
            j0 = 2 * g
            fire(j0 + 1, 1)
            drain_scatter(j0, 0)

            @pl.when(g < npairs - 1)
            def _():
                fire(j0 + 2, 0)

            drain_scatter(j0 + 1, 1)
            return carry

        lax.fori_loop(0, npairs, body, 0)

    @pl.when(wid < NCHUNKS - BASE_CH * NUM_WORKERS)
    def _():
        fire(BASE_CH - HALF, 0)
        drain_scatter(BASE_CH - HALF, 0)

    plsc.subcore_barrier()

    pltpu.sync_copy(acc_sh.at[pl.ds(row0, ROWS_PER_TILE)],
                    out_hbm.at[cid, pl.ds(row0, ROWS_PER_TILE)])


def _add_body(a_ref, b_ref, o_ref):
    o_ref[...] = a_ref[0] + b_ref[0]


_ADD_BLOCK = 1000


def _combine(partial):
    return pl.pallas_call(
        _add_body,
        grid=(N_NODES // _ADD_BLOCK,),
        in_specs=[
            pl.BlockSpec((1, _ADD_BLOCK, D_FEAT), lambda i: (0, i, 0)),
            pl.BlockSpec((1, _ADD_BLOCK, D_FEAT), lambda i: (1, i, 0)),
        ],
        out_specs=pl.BlockSpec((_ADD_BLOCK, D_FEAT), lambda i: (i, 0)),
        out_shape=jax.ShapeDtypeStruct((N_NODES, D_FEAT), jnp.float32),
    )(partial, partial)


@jax.jit
def kernel(edge_index, x):
    dst = edge_index[0].reshape(NCHUNKS, CHUNK)
    src = edge_index[1].reshape(NCHUNKS, CHUNK)
    zeros = jnp.zeros((ROWS_PER_TILE, D_FEAT), jnp.float32)
    partial = _mp_scatter(src, dst, x, zeros)
    return _combine(partial)


# speedup vs baseline: 1.0307x; 1.0307x over previous
import functools

import jax
import jax.numpy as jnp
from jax import lax
from jax.experimental import pallas as pl
from jax.experimental.pallas import tpu as pltpu
from jax.experimental.pallas import tpu_sc as plsc

N_NODES = 10000
N_EDGES = 320000
D_FEAT = 128

NUM_CORES = 2
NUM_SUBCORES = 16
NUM_WORKERS = NUM_CORES * NUM_SUBCORES

CHUNK = 128
NCHUNKS = N_EDGES // CHUNK
BASE_CH = NCHUNKS // NUM_WORKERS
HALF = 40
N_PAD = 10240
ROWS_PER_TILE = N_PAD // NUM_SUBCORES

_mesh = plsc.VectorSubcoreMesh(core_axis_name="c", subcore_axis_name="s")


@functools.partial(
    pl.kernel,
    mesh=_mesh,
    out_type=jax.ShapeDtypeStruct((NUM_CORES, N_PAD, D_FEAT), jnp.float32),
    scratch_types=[
        pltpu.VMEM((2 * HALF,), jnp.int32),
        pltpu.VMEM((HALF, CHUNK), jnp.int32),
        pltpu.VMEM((HALF, CHUNK), jnp.int32),
        pltpu.VMEM((2, CHUNK, D_FEAT), jnp.float32),
        pltpu.VMEM_SHARED((N_PAD, D_FEAT), jnp.float32),
        pltpu.SemaphoreType.DMA,
        pltpu.SemaphoreType.DMA,
        pltpu.SemaphoreType.DMA,
    ],
)
def _mp_scatter(src_hbm, dst_hbm, x_hbm, zeros_hbm, out_hbm,
                ilist_v, src_v, dst_v, rows_v, acc_sh, sem0, sem1, semz):
    cid = lax.axis_index("c")
    sid = lax.axis_index("s")
    wid = sid * NUM_CORES + cid
    row0 = sid * ROWS_PER_TILE

    pltpu.async_copy(zeros_hbm, acc_sh.at[pl.ds(row0, ROWS_PER_TILE)], semz)

    lane = lax.iota(jnp.int32, 16)
    for g in range(2 * HALF // 16):
        ilist_v[pl.ds(16 * g, 16)] = jnp.minimum(
            wid + 32 * (16 * g) + 32 * lane, NCHUNKS - 1)

    sems = (sem0, sem1)

    def fire(j, b):
        pltpu.async_copy(x_hbm.at[src_v.at[j]], rows_v.at[b], sems[b])

    def drain_scatter(j, b):
        pltpu.make_async_copy(x_hbm.at[src_v.at[j]], rows_v.at[b], sems[b]).wait()
        pltpu.sync_copy(rows_v.at[b], acc_sh.at[dst_v.at[j]], add=True)

    for h in range(2):
        pltpu.sync_copy(src_hbm.at[ilist_v.at[pl.ds(h * HALF, HALF)]], src_v)
        pltpu.sync_copy(dst_hbm.at[ilist_v.at[pl.ds(h * HALF, HALF)]], dst_v)

        fire(0, 0)

        if h == 0:
            pltpu.make_async_copy(
                zeros_hbm, acc_sh.at[pl.ds(row0, ROWS_PER_TILE)], semz).wait()
            plsc.subcore_barrier()
            npairs = HALF // 2
        else:
            npairs = (BASE_CH - HALF) // 2

        def body(g, carry):
            j0 = 2 * g
            fire(j0 + 1, 1)
            drain_scatter(j0, 0)

            @pl.when(g < npairs - 1)
            def _():
                fire(j0 + 2, 0)

            drain_scatter(j0 + 1, 1)
            return carry

        lax.fori_loop(0, npairs, body, 0)

    @pl.when(wid < NCHUNKS - BASE_CH * NUM_WORKERS)
    def _():
        fire(BASE_CH - HALF, 0)
        drain_scatter(BASE_CH - HALF, 0)

    plsc.subcore_barrier()

    pltpu.sync_copy(acc_sh.at[pl.ds(row0, ROWS_PER_TILE)],
                    out_hbm.at[cid, pl.ds(row0, ROWS_PER_TILE)])


def _add_body(a_ref, b_ref, o_ref):
    o_ref[...] = a_ref[0] + b_ref[0]


_ADD_BLOCK = 2000


def _combine(partial):
    return pl.pallas_call(
        _add_body,
        grid=(N_NODES // _ADD_BLOCK,),
        in_specs=[
            pl.BlockSpec((1, _ADD_BLOCK, D_FEAT), lambda i: (0, i, 0)),
            pl.BlockSpec((1, _ADD_BLOCK, D_FEAT), lambda i: (1, i, 0)),
        ],
        out_specs=pl.BlockSpec((_ADD_BLOCK, D_FEAT), lambda i: (i, 0)),
        out_shape=jax.ShapeDtypeStruct((N_NODES, D_FEAT), jnp.float32),
    )(partial, partial)


@jax.jit
def kernel(edge_index, x):
    dst = edge_index[0].reshape(NCHUNKS, CHUNK)
    src = edge_index[1].reshape(NCHUNKS, CHUNK)
    zeros = jnp.zeros((ROWS_PER_TILE, D_FEAT), jnp.float32)
    partial = _mp_scatter(src, dst, x, zeros)
    return _combine(partial)


# speedup vs baseline: 1.0337x; 1.0029x over previous
import functools

import jax
import jax.numpy as jnp
from jax import lax
from jax.experimental import pallas as pl
from jax.experimental.pallas import tpu as pltpu
from jax.experimental.pallas import tpu_sc as plsc

N_NODES = 10000
N_EDGES = 320000
D_FEAT = 128

NUM_CORES = 2
NUM_SUBCORES = 16
NUM_WORKERS = NUM_CORES * NUM_SUBCORES

CHUNK = 128
NCHUNKS = N_EDGES // CHUNK
BASE_CH = NCHUNKS // NUM_WORKERS
HALF = 40
N_PAD = 10240
ROWS_PER_TILE = N_PAD // NUM_SUBCORES

_mesh = plsc.VectorSubcoreMesh(core_axis_name="c", subcore_axis_name="s")


@functools.partial(
    pl.kernel,
    mesh=_mesh,
    out_type=jax.ShapeDtypeStruct((NUM_CORES, N_PAD, D_FEAT), jnp.float32),
    scratch_types=[
        pltpu.VMEM((2 * HALF,), jnp.int32),
        pltpu.VMEM((HALF, CHUNK), jnp.int32),
        pltpu.VMEM((HALF, CHUNK), jnp.int32),
        pltpu.VMEM((HALF, CHUNK), jnp.int32),
        pltpu.VMEM((2, CHUNK, D_FEAT), jnp.float32),
        pltpu.VMEM_SHARED((N_PAD, D_FEAT), jnp.float32),
        pltpu.SemaphoreType.DMA,
        pltpu.SemaphoreType.DMA,
        pltpu.SemaphoreType.DMA,
        pltpu.SemaphoreType.DMA,
    ],
)
def _mp_scatter(src_hbm, dst_hbm, x_hbm, zeros_hbm, out_hbm,
                ilist_v, src0_v, src1_v, dst_v, rows_v, acc_sh,
                sem0, sem1, semz, semp):
    cid = lax.axis_index("c")
    sid = lax.axis_index("s")
    wid = sid * NUM_CORES + cid
    row0 = sid * ROWS_PER_TILE

    pltpu.async_copy(zeros_hbm, acc_sh.at[pl.ds(row0, ROWS_PER_TILE)], semz)

    lane = lax.iota(jnp.int32, 16)
    for g in range(2 * HALF // 16):
        ilist_v[pl.ds(16 * g, 16)] = jnp.minimum(
            wid + 32 * (16 * g) + 32 * lane, NCHUNKS - 1)

    sems = (sem0, sem1)

    def make_fire_drain(src_v):
        def fire(j, b):
            pltpu.async_copy(x_hbm.at[src_v.at[j]], rows_v.at[b], sems[b])

        def drain_scatter(j, b):
            pltpu.make_async_copy(
                x_hbm.at[src_v.at[j]], rows_v.at[b], sems[b]).wait()
            pltpu.sync_copy(rows_v.at[b], acc_sh.at[dst_v.at[j]], add=True)

        return fire, drain_scatter

    for h in range(2):
        src_v = (src0_v, src1_v)[h]
        fire, drain_scatter = make_fire_drain(src_v)
        if h == 0:
            pltpu.sync_copy(src_hbm.at[ilist_v.at[pl.ds(0, HALF)]], src0_v)
            pltpu.async_copy(
                src_hbm.at[ilist_v.at[pl.ds(HALF, HALF)]], src1_v, semp)
        else:
            pltpu.make_async_copy(
                src_hbm.at[ilist_v.at[pl.ds(HALF, HALF)]], src1_v, semp).wait()
        pltpu.sync_copy(dst_hbm.at[ilist_v.at[pl.ds(h * HALF, HALF)]], dst_v)

        fire(0, 0)

        if h == 0:
            pltpu.make_async_copy(
                zeros_hbm, acc_sh.at[pl.ds(row0, ROWS_PER_TILE)], semz).wait()
            plsc.subcore_barrier()
            npairs = HALF // 2
        else:
            npairs = (BASE_CH - HALF) // 2

        def body(g, carry):
            j0 = 2 * g
            fire(j0 + 1, 1)
            drain_scatter(j0, 0)

            @pl.when(g < npairs - 1)
            def _():
                fire(j0 + 2, 0)

            drain_scatter(j0 + 1, 1)
            return carry

        lax.fori_loop(0, npairs, body, 0)

    @pl.when(wid < NCHUNKS - BASE_CH * NUM_WORKERS)
    def _():
        fire(BASE_CH - HALF, 0)
        drain_scatter(BASE_CH - HALF, 0)

    plsc.subcore_barrier()

    pltpu.sync_copy(acc_sh.at[pl.ds(row0, ROWS_PER_TILE)],
                    out_hbm.at[cid, pl.ds(row0, ROWS_PER_TILE)])


def _add_body(a_ref, b_ref, o_ref):
    o_ref[...] = a_ref[0] + b_ref[0]


_ADD_BLOCK = 2000


def _combine(partial):
    return pl.pallas_call(
        _add_body,
        grid=(N_NODES // _ADD_BLOCK,),
        in_specs=[
            pl.BlockSpec((1, _ADD_BLOCK, D_FEAT), lambda i: (0, i, 0)),
            pl.BlockSpec((1, _ADD_BLOCK, D_FEAT), lambda i: (1, i, 0)),
        ],
        out_specs=pl.BlockSpec((_ADD_BLOCK, D_FEAT), lambda i: (i, 0)),
        out_shape=jax.ShapeDtypeStruct((N_NODES, D_FEAT), jnp.float32),
    )(partial, partial)


@jax.jit
def kernel(edge_index, x):
    dst = edge_index[0].reshape(NCHUNKS, CHUNK)
    src = edge_index[1].reshape(NCHUNKS, CHUNK)
    zeros = jnp.zeros((ROWS_PER_TILE, D_FEAT), jnp.float32)
    partial = _mp_scatter(src, dst, x, zeros)
    return _combine(partial)
